# baseline (speedup 1.0000x reference)
import jax
import jax.numpy as jnp
from jax import lax
from jax.experimental import pallas as pl
from jax.experimental.pallas import tpu as pltpu

N_CAT = 238
D_IMG = 3 * 32 * 32
BM = 512
BN = 512


def _mm_body(wt_ref, xt_ref, bt_ref, logpt_ref, ent_ref, kl_ref):
    wt = wt_ref[...]
    xt = xt_ref[...]
    logits = jnp.dot(wt, xt, preferred_element_type=jnp.float32) + bt_ref[...]
    m = jnp.max(logits, axis=0, keepdims=True)
    sh = logits - m
    lse = jnp.log(jnp.sum(jnp.exp(sh), axis=0, keepdims=True))
    logp = sh - lse
    p = jnp.exp(logp)
    logpt_ref[...] = logp
    ent_ref[...] = -jnp.sum(p * logp, axis=0, keepdims=True)
    kl_ref[...] = jnp.sum(p * (logp - jnp.log(1.0 / N_CAT)), axis=0,
                          keepdims=True)


def _mm_call(wt, xt, bt):
    bsz = xt.shape[1]
    grid = (bsz // BN,)
    return pl.pallas_call(
        _mm_body,
        grid=grid,
        in_specs=[
            pl.BlockSpec((N_CAT, D_IMG), lambda i: (0, 0)),
            pl.BlockSpec((D_IMG, BN), lambda i: (0, i)),
            pl.BlockSpec((N_CAT, 1), lambda i: (0, 0)),
        ],
        out_specs=[
            pl.BlockSpec((N_CAT, BN), lambda i: (0, i)),
            pl.BlockSpec((1, BN), lambda i: (0, i)),
            pl.BlockSpec((1, BN), lambda i: (0, i)),
        ],
        out_shape=[
            jax.ShapeDtypeStruct((N_CAT, bsz), jnp.float32),
            jax.ShapeDtypeStruct((1, bsz), jnp.float32),
            jax.ShapeDtypeStruct((1, bsz), jnp.float32),
        ],
        compiler_params=pltpu.CompilerParams(
            dimension_semantics=("parallel",),
        ),
    )(wt, xt, bt)


def _samp_body(logp_ref, u_ref, samp_ref, slp_ref):
    n_copies = u_ref.shape[0]
    logp = logp_ref[...]
    iota = lax.broadcasted_iota(jnp.int32, (BM, N_CAT), 1)
    for k in range(n_copies):
        g = -jnp.log(-jnp.log(u_ref[k]))
        s = jnp.argmax(logp + g, axis=-1).astype(jnp.int32)
        samp_ref[k, :] = s
        slp_ref[k, :] = jnp.sum(jnp.where(iota == s[:, None], logp, 0.0), axis=-1)


def _samp_call(logp_row, u):
    bsz = logp_row.shape[0]
    n_copies = u.shape[0]
    grid = (bsz // BM,)
    return pl.pallas_call(
        _samp_body,
        grid=grid,
        in_specs=[
            pl.BlockSpec((BM, N_CAT), lambda i: (i, 0)),
            pl.BlockSpec((n_copies, BM, N_CAT), lambda i: (0, i, 0)),
        ],
        out_specs=[
            pl.BlockSpec((n_copies, BM), lambda i: (0, i)),
            pl.BlockSpec((n_copies, BM), lambda i: (0, i)),
        ],
        out_shape=[
            jax.ShapeDtypeStruct((n_copies, bsz), jnp.int32),
            jax.ShapeDtypeStruct((n_copies, bsz), jnp.float32),
        ],
        compiler_params=pltpu.CompilerParams(
            dimension_semantics=("parallel",),
        ),
    )(logp_row, u)


_BF = 512
_BS = 8192


def _onehot_body(bank_t_ref, samp_ref, out_ref):
    oh = (lax.broadcasted_iota(jnp.int32, (N_CAT, _BS), 0)
          == samp_ref[...]).astype(jnp.float32)
    out_ref[...] = jnp.dot(bank_t_ref[...], oh,
                           preferred_element_type=jnp.float32)


def _onehot_gather_call(bank_t, idx_row, n_rows, f_rows):
    grid = (f_rows // _BF, n_rows // _BS)
    return pl.pallas_call(
        _onehot_body,
        grid=grid,
        in_specs=[
            pl.BlockSpec((_BF, N_CAT), lambda fi, si: (fi, 0)),
            pl.BlockSpec((1, _BS), lambda fi, si: (0, si)),
        ],
        out_specs=pl.BlockSpec((_BF, _BS), lambda fi, si: (fi, si)),
        out_shape=jax.ShapeDtypeStruct((f_rows, n_rows), jnp.float32),
        compiler_params=pltpu.CompilerParams(
            dimension_semantics=("parallel", "parallel"),
        ),
    )(bank_t, idx_row)


def kernel(x, u, W, b, bank, n_copies):
    bsz = x.shape[0]
    n_copies_static = u.shape[0]
    n_rows = n_copies_static * bsz
    xt = x.reshape(bsz, -1).T
    logpt, ent, kl = _mm_call(W.T, xt, b.reshape(-1, 1))
    samp, slp = _samp_call(logpt.T, u)
    idx_row = samp.reshape(1, n_rows)
    bank_t = bank.reshape(N_CAT, D_IMG).T
    out_t = _onehot_gather_call(bank_t, idx_row, n_rows, D_IMG)
    c, h, w = bank.shape[1:]
    x_out = jax.lax.stop_gradient(
        out_t.reshape(c, h, w, n_rows).transpose(3, 0, 1, 2)
    )
    return (x_out, slp.reshape(-1), ent.reshape(-1), kl.reshape(-1))


# speedup vs baseline: 1.0113x; 1.0113x over previous
import jax
import jax.numpy as jnp
from jax import lax
from jax.experimental import pallas as pl
from jax.experimental.pallas import tpu as pltpu

N_CAT = 238
D_IMG = 3 * 32 * 32
BN = 512
_BF = 512
_BS = 4096


def _mm_body(wt_ref, xt_ref, bt_ref, logpt_ref, ent_ref, kl_ref):
    wt = wt_ref[...]
    xt = xt_ref[...]
    logits = jnp.dot(wt, xt, preferred_element_type=jnp.float32) + bt_ref[...]
    m = jnp.max(logits, axis=0, keepdims=True)
    sh = logits - m
    lse = jnp.log(jnp.sum(jnp.exp(sh), axis=0, keepdims=True))
    logp = sh - lse
    p = jnp.exp(logp)
    logpt_ref[...] = logp
    ent_ref[...] = -jnp.sum(p * logp, axis=0, keepdims=True)
    kl_ref[...] = jnp.sum(p * (logp - jnp.log(1.0 / N_CAT)), axis=0,
                          keepdims=True)


def _mm_call(wt, xt, bt):
    bsz = xt.shape[1]
    grid = (bsz // BN,)
    return pl.pallas_call(
        _mm_body,
        grid=grid,
        in_specs=[
            pl.BlockSpec((N_CAT, D_IMG), lambda i: (0, 0)),
            pl.BlockSpec((D_IMG, BN), lambda i: (0, i)),
            pl.BlockSpec((N_CAT, 1), lambda i: (0, 0)),
        ],
        out_specs=[
            pl.BlockSpec((N_CAT, BN), lambda i: (0, i)),
            pl.BlockSpec((1, BN), lambda i: (0, i)),
            pl.BlockSpec((1, BN), lambda i: (0, i)),
        ],
        out_shape=[
            jax.ShapeDtypeStruct((N_CAT, bsz), jnp.float32),
            jax.ShapeDtypeStruct((1, bsz), jnp.float32),
            jax.ShapeDtypeStruct((1, bsz), jnp.float32),
        ],
        compiler_params=pltpu.CompilerParams(
            dimension_semantics=("parallel",),
        ),
    )(wt, xt, bt)


def _fused_body(bank_t_ref, logp_ref, u_ref, out_ref, samp_ref, slp_ref,
                sidx_ref):
    fi = pl.program_id(1)

    @pl.when(fi == 0)
    def _sample():
        logp = logp_ref[...]
        u = u_ref[0]
        g = -jnp.log(-jnp.log(u))
        s = jnp.argmax(logp + g, axis=-1).astype(jnp.int32)
        sidx_ref[0, :] = s
        samp_ref[0, 0, :] = s
        iota = lax.broadcasted_iota(jnp.int32, (_BS, N_CAT), 1)
        slp_ref[0, 0, :] = jnp.sum(
            jnp.where(iota == s[:, None], logp, 0.0), axis=-1)

    oh = (lax.broadcasted_iota(jnp.int32, (N_CAT, _BS), 0)
          == sidx_ref[...]).astype(jnp.float32)
    out_ref[...] = jnp.dot(bank_t_ref[...], oh,
                           preferred_element_type=jnp.float32)


def _fused_call(bank_t, logp_row, u):
    n_copies, bsz, _ = u.shape
    n_rows = n_copies * bsz
    grid = (n_copies, D_IMG // _BF)
    return pl.pallas_call(
        _fused_body,
        grid=grid,
        in_specs=[
            pl.BlockSpec((_BF, N_CAT), lambda si, fi: (fi, 0)),
            pl.BlockSpec((_BS, N_CAT), lambda si, fi: (0, 0)),
            pl.BlockSpec((1, _BS, N_CAT), lambda si, fi: (si, 0, 0)),
        ],
        out_specs=[
            pl.BlockSpec((_BF, _BS), lambda si, fi: (fi, si)),
            pl.BlockSpec((1, 1, _BS), lambda si, fi: (si, 0, 0)),
            pl.BlockSpec((1, 1, _BS), lambda si, fi: (si, 0, 0)),
        ],
        out_shape=[
            jax.ShapeDtypeStruct((D_IMG, n_rows), jnp.float32),
            jax.ShapeDtypeStruct((n_copies, 1, bsz), jnp.int32),
            jax.ShapeDtypeStruct((n_copies, 1, bsz), jnp.float32),
        ],
        scratch_shapes=[pltpu.VMEM((1, _BS), jnp.int32)],
        compiler_params=pltpu.CompilerParams(
            dimension_semantics=("arbitrary", "arbitrary"),
        ),
    )(bank_t, logp_row, u)


def kernel(x, u, W, b, bank, n_copies):
    bsz = x.shape[0]
    n_copies_static = u.shape[0]
    n_rows = n_copies_static * bsz
    xt = x.reshape(bsz, -1).T
    logpt, ent, kl = _mm_call(W.T, xt, b.reshape(-1, 1))
    bank_t = bank.reshape(N_CAT, D_IMG).T
    out_t, samp, slp = _fused_call(bank_t, logpt.T, u)
    c, h, w = bank.shape[1:]
    x_out = jax.lax.stop_gradient(
        out_t.reshape(c, h, w, n_rows).transpose(3, 0, 1, 2)
    )
    return (x_out, slp.reshape(-1), ent.reshape(-1), kl.reshape(-1))


# speedup vs baseline: 1.0217x; 1.0103x over previous
import jax
import jax.numpy as jnp
from jax import lax
from jax.experimental import pallas as pl
from jax.experimental.pallas import tpu as pltpu

N_CAT = 238
D_IMG = 3 * 32 * 32
BN = 512
_BF = 512
_BS = 4096


def _mm_body(wt_ref, xt_ref, bt_ref, logpt_ref, ent_ref, kl_ref):
    wt = wt_ref[...]
    xt = xt_ref[...]
    logits = jnp.dot(wt, xt, preferred_element_type=jnp.float32) + bt_ref[...]
    m = jnp.max(logits, axis=0, keepdims=True)
    sh = logits - m
    lse = jnp.log(jnp.sum(jnp.exp(sh), axis=0, keepdims=True))
    logp = sh - lse
    p = jnp.exp(logp)
    logpt_ref[...] = logp
    ent_ref[...] = -jnp.sum(p * logp, axis=0, keepdims=True)
    kl_ref[...] = jnp.sum(p * (logp - jnp.log(1.0 / N_CAT)), axis=0,
                          keepdims=True)


def _mm_call(wt, xt, bt):
    bsz = xt.shape[1]
    grid = (bsz // BN,)
    return pl.pallas_call(
        _mm_body,
        grid=grid,
        in_specs=[
            pl.BlockSpec((N_CAT, D_IMG), lambda i: (0, 0)),
            pl.BlockSpec((D_IMG, BN), lambda i: (0, i)),
            pl.BlockSpec((N_CAT, 1), lambda i: (0, 0)),
        ],
        out_specs=[
            pl.BlockSpec((N_CAT, BN), lambda i: (0, i)),
            pl.BlockSpec((1, BN), lambda i: (0, i)),
            pl.BlockSpec((1, BN), lambda i: (0, i)),
        ],
        out_shape=[
            jax.ShapeDtypeStruct((N_CAT, bsz), jnp.float32),
            jax.ShapeDtypeStruct((1, bsz), jnp.float32),
            jax.ShapeDtypeStruct((1, bsz), jnp.float32),
        ],
        compiler_params=pltpu.CompilerParams(
            dimension_semantics=("parallel",),
        ),
    )(wt, xt, bt)


def _fused_body(bank_t_ref, logp_ref, u_hbm, out_ref, samp_ref, slp_ref,
                sidx_ref, u_v, u_sem):
    si = pl.program_id(0)
    fi = pl.program_id(1)
    n_si = pl.num_programs(0)

    @pl.when(jnp.logical_and(si == 0, fi == 0))
    def _prefetch_first():
        pltpu.make_async_copy(u_hbm.at[0], u_v, u_sem).start()

    @pl.when(fi == 0)
    def _sample():
        logp = logp_ref[...]
        pltpu.make_async_copy(u_hbm.at[si], u_v, u_sem).wait()
        g = -jnp.log(-jnp.log(u_v[...]))
        s = jnp.argmax(logp + g, axis=-1).astype(jnp.int32)
        sidx_ref[0, :] = s
        samp_ref[0, 0, :] = s
        iota = lax.broadcasted_iota(jnp.int32, (_BS, N_CAT), 1)
        slp_ref[0, 0, :] = jnp.sum(
            jnp.where(iota == s[:, None], logp, 0.0), axis=-1)

    @pl.when(jnp.logical_and(fi == 1, si + 1 < n_si))
    def _prefetch_next():
        pltpu.make_async_copy(u_hbm.at[si + 1], u_v, u_sem).start()

    oh = (lax.broadcasted_iota(jnp.int32, (N_CAT, _BS), 0)
          == sidx_ref[...]).astype(jnp.float32)
    out_ref[...] = jnp.dot(bank_t_ref[...], oh,
                           preferred_element_type=jnp.float32)


def _fused_call(bank_t, logp_row, u):
    n_copies, bsz, _ = u.shape
    n_rows = n_copies * bsz
    grid = (n_copies, D_IMG // _BF)
    return pl.pallas_call(
        _fused_body,
        grid=grid,
        in_specs=[
            pl.BlockSpec((_BF, N_CAT), lambda si, fi: (fi, 0)),
            pl.BlockSpec((_BS, N_CAT), lambda si, fi: (0, 0)),
            pl.BlockSpec(memory_space=pl.ANY),
        ],
        out_specs=[
            pl.BlockSpec((_BF, _BS), lambda si, fi: (fi, si)),
            pl.BlockSpec((1, 1, _BS), lambda si, fi: (si, 0, 0)),
            pl.BlockSpec((1, 1, _BS), lambda si, fi: (si, 0, 0)),
        ],
        out_shape=[
            jax.ShapeDtypeStruct((D_IMG, n_rows), jnp.float32),
            jax.ShapeDtypeStruct((n_copies, 1, bsz), jnp.int32),
            jax.ShapeDtypeStruct((n_copies, 1, bsz), jnp.float32),
        ],
        scratch_shapes=[
            pltpu.VMEM((1, _BS), jnp.int32),
            pltpu.VMEM((_BS, N_CAT), jnp.float32),
            pltpu.SemaphoreType.DMA,
        ],
        compiler_params=pltpu.CompilerParams(
            dimension_semantics=("arbitrary", "arbitrary"),
        ),
    )(bank_t, logp_row, u)


def kernel(x, u, W, b, bank, n_copies):
    bsz = x.shape[0]
    n_copies_static = u.shape[0]
    n_rows = n_copies_static * bsz
    xt = x.reshape(bsz, -1).T
    logpt, ent, kl = _mm_call(W.T, xt, b.reshape(-1, 1))
    bank_t = bank.reshape(N_CAT, D_IMG).T
    out_t, samp, slp = _fused_call(bank_t, logpt.T, u)
    c, h, w = bank.shape[1:]
    x_out = jax.lax.stop_gradient(
        out_t.reshape(c, h, w, n_rows).transpose(3, 0, 1, 2)
    )
    return (x_out, slp.reshape(-1), ent.reshape(-1), kl.reshape(-1))


# speedup vs baseline: 1.0362x; 1.0141x over previous
import jax
import jax.numpy as jnp
from jax import lax
from jax.experimental import pallas as pl
from jax.experimental.pallas import tpu as pltpu

N_CAT = 238
D_IMG = 3 * 32 * 32
BN = 512
_BF = 512
_BS = 4096


def _mm_body(wt_ref, xt_ref, bt_ref, logpt_ref, ent_ref, kl_ref):
    wt = wt_ref[...]
    xt = xt_ref[...]
    logits = jnp.dot(wt, xt, preferred_element_type=jnp.float32) + bt_ref[...]
    m = jnp.max(logits, axis=0, keepdims=True)
    sh = logits - m
    lse = jnp.log(jnp.sum(jnp.exp(sh), axis=0, keepdims=True))
    logp = sh - lse
    p = jnp.exp(logp)
    logpt_ref[...] = logp
    ent_ref[...] = -jnp.sum(p * logp, axis=0, keepdims=True)
    kl_ref[...] = jnp.sum(p * (logp - jnp.log(1.0 / N_CAT)), axis=0,
                          keepdims=True)


def _mm_call(wt, xt, bt):
    bsz = xt.shape[1]
    grid = (bsz // BN,)
    return pl.pallas_call(
        _mm_body,
        grid=grid,
        in_specs=[
            pl.BlockSpec((N_CAT, D_IMG), lambda i: (0, 0)),
            pl.BlockSpec((D_IMG, BN), lambda i: (0, i)),
            pl.BlockSpec((N_CAT, 1), lambda i: (0, 0)),
        ],
        out_specs=[
            pl.BlockSpec((N_CAT, BN), lambda i: (0, i)),
            pl.BlockSpec((1, BN), lambda i: (0, i)),
            pl.BlockSpec((1, BN), lambda i: (0, i)),
        ],
        out_shape=[
            jax.ShapeDtypeStruct((N_CAT, bsz), jnp.float32),
            jax.ShapeDtypeStruct((1, bsz), jnp.float32),
            jax.ShapeDtypeStruct((1, bsz), jnp.float32),
        ],
        compiler_params=pltpu.CompilerParams(
            dimension_semantics=("parallel",),
        ),
    )(wt, xt, bt)


_GCH = 4


def _fused_body(bank_t_ref, logp_ref, u_hbm, out_ref, samp_ref, slp_ref,
                sidx_ref, g_v, u_v0, u_v1, us0, us1):
    si = pl.program_id(0)
    fi = pl.program_id(1)
    n_si = pl.num_programs(0)
    u_bufs = ((u_v0, us0), (u_v1, us1))
    rows = _BS // _GCH

    @pl.when(jnp.logical_and(si == 0, fi == 0))
    def _first_block():
        pltpu.make_async_copy(u_hbm.at[0], u_v0, us0).start()
        pltpu.make_async_copy(u_hbm.at[0], u_v0, us0).wait()
        g_v[...] = -jnp.log(-jnp.log(u_v0[...]))

    @pl.when(fi == 0)
    def _sample():
        logp = logp_ref[...]
        s = jnp.argmax(logp + g_v[...], axis=-1).astype(jnp.int32)
        sidx_ref[0, :] = s
        samp_ref[0, 0, :] = s
        iota = lax.broadcasted_iota(jnp.int32, (_BS, N_CAT), 1)
        slp_ref[0, 0, :] = jnp.sum(
            jnp.where(iota == s[:, None], logp, 0.0), axis=-1)

    nsi = si + 1
    have_next = nsi < n_si

    @pl.when(jnp.logical_and(fi == 1, have_next))
    def _prefetch_next():
        buf, sem = u_bufs[1]
        pltpu.make_async_copy(u_hbm.at[nsi], buf, sem).start()

    for c in range(_GCH):
        @pl.when(jnp.logical_and(fi == 2 + c, have_next))
        def _gumbel_chunk(c=c):
            buf, sem = u_bufs[1]
            if c == 0:
                pltpu.make_async_copy(u_hbm.at[nsi], buf, sem).wait()
            sl = pl.ds(c * rows, rows)
            g_v[sl, :] = -jnp.log(-jnp.log(buf[sl, :]))

    oh = (lax.broadcasted_iota(jnp.int32, (N_CAT, _BS), 0)
          == sidx_ref[...]).astype(jnp.float32)
    out_ref[...] = jnp.dot(bank_t_ref[...], oh,
                           preferred_element_type=jnp.float32)


def _fused_call(bank_t, logp_row, u):
    n_copies, bsz, _ = u.shape
    n_rows = n_copies * bsz
    grid = (n_copies, D_IMG // _BF)
    return pl.pallas_call(
        _fused_body,
        grid=grid,
        in_specs=[
            pl.BlockSpec((_BF, N_CAT), lambda si, fi: (fi, 0)),
            pl.BlockSpec((_BS, N_CAT), lambda si, fi: (0, 0)),
            pl.BlockSpec(memory_space=pl.ANY),
        ],
        out_specs=[
            pl.BlockSpec((_BF, _BS), lambda si, fi: (fi, si)),
            pl.BlockSpec((1, 1, _BS), lambda si, fi: (si, 0, 0)),
            pl.BlockSpec((1, 1, _BS), lambda si, fi: (si, 0, 0)),
        ],
        out_shape=[
            jax.ShapeDtypeStruct((D_IMG, n_rows), jnp.float32),
            jax.ShapeDtypeStruct((n_copies, 1, bsz), jnp.int32),
            jax.ShapeDtypeStruct((n_copies, 1, bsz), jnp.float32),
        ],
        scratch_shapes=[
            pltpu.VMEM((1, _BS), jnp.int32),
            pltpu.VMEM((_BS, N_CAT), jnp.float32),
            pltpu.VMEM((_BS, N_CAT), jnp.float32),
            pltpu.VMEM((_BS, N_CAT), jnp.float32),
            pltpu.SemaphoreType.DMA,
            pltpu.SemaphoreType.DMA,
        ],
        compiler_params=pltpu.CompilerParams(
            dimension_semantics=("arbitrary", "arbitrary"),
        ),
    )(bank_t, logp_row, u)


def kernel(x, u, W, b, bank, n_copies):
    bsz = x.shape[0]
    n_copies_static = u.shape[0]
    n_rows = n_copies_static * bsz
    xt = x.reshape(bsz, -1).T
    logpt, ent, kl = _mm_call(W.T, xt, b.reshape(-1, 1))
    bank_t = bank.reshape(N_CAT, D_IMG).T
    out_t, samp, slp = _fused_call(bank_t, logpt.T, u)
    c, h, w = bank.shape[1:]
    x_out = jax.lax.stop_gradient(
        out_t.reshape(c, h, w, n_rows).transpose(3, 0, 1, 2)
    )
    return (x_out, slp.reshape(-1), ent.reshape(-1), kl.reshape(-1))
